# baseline (speedup 1.0000x reference)
import functools

import jax
import jax.numpy as jnp
from jax import lax
from jax.experimental import pallas as pl
from jax.experimental.pallas import tpu as pltpu
from jax.experimental.pallas import tpu_sc as plsc

N = 10000
NP = 12288
E = 320000
D_IN = 128
DH = 8
DP = 8
ZDP = 16
NC, NS, L = 2, 16, 16
NW = NC * NS
EPW = E // NW
NPW = NP // NW
ECH = 125
NCH = EPW // ECH
RPS = NP // NS
NR = NP // 128
RPT = NR // NS
NBUF = 4

_MESH = plsc.VectorSubcoreMesh(core_axis_name="c", subcore_axis_name="s")
_SC_PARAMS = pltpu.CompilerParams(needs_layout_passes=False,
                                  use_tc_tiling_on_sc=False)


def _wid():
    return lax.axis_index("s") * NC + lax.axis_index("c")


def _rsqrt_newton(x):
    i = lax.bitcast_convert_type(x, jnp.int32)
    y = lax.bitcast_convert_type(jnp.int32(0x5F3759DF) - (i >> 1), jnp.float32)
    for _ in range(3):
        y = y * (1.5 - 0.5 * x * y * y)
    return y


def _fill_iota_row(ref, row, offset):
    ioa = lax.broadcasted_iota(jnp.int32, (L,), 0)
    for k in range(NR // L):
        ref[row, pl.ds(k * L, L)] = ioa + (offset + k * L)


def _mm_body(x_ref, w_ref, o_ref):
    o_ref[...] = jnp.dot(x_ref[...], w_ref[...],
                         preferred_element_type=jnp.float32)


def _matmul(xp, w1p):
    return pl.pallas_call(
        _mm_body,
        grid=(NP // 1024,),
        in_specs=[
            pl.BlockSpec((1024, D_IN), lambda i: (i, 0)),
            pl.BlockSpec((D_IN, ZDP), lambda i: (0, 0)),
        ],
        out_specs=pl.BlockSpec((1024, ZDP), lambda i: (i, 0)),
        out_shape=jax.ShapeDtypeStruct((NP, ZDP), jnp.float32),
    )(xp, w1p)


def _deg_body(srcw, dstw, degs_p, degd_p, idxv, h_out, h_in, iotab, hS):
    c = lax.axis_index("c")
    s = lax.axis_index("s")
    w = _wid()
    zero = jnp.zeros((L,), jnp.float32)

    def zbody(r, _):
        for k in range(128 // L):
            h_out[r, pl.ds(k * L, L)] = zero
            h_in[r, pl.ds(k * L, L)] = zero
        return 0

    lax.fori_loop(0, NR, zbody, 0, unroll=2)
    _fill_iota_row(iotab, 0, 0)
    _fill_iota_row(iotab, 1, NR)

    one = jnp.full((L,), 1.0, jnp.float32)
    pltpu.sync_copy(srcw.at[w], idxv)

    def sbody(i, _):
        d = idxv[pl.ds(i * L, L)]
        plsc.addupdate_scatter(h_out, [d >> 7, d & 127], one)
        return 0

    lax.fori_loop(0, EPW // L, sbody, 0, unroll=8)
    pltpu.sync_copy(dstw.at[w], idxv)

    def dbody(i, _):
        d = idxv[pl.ds(i * L, L)]
        plsc.addupdate_scatter(h_in, [d >> 7, d & 127], one)
        return 0

    lax.fori_loop(0, EPW // L, dbody, 0, unroll=8)

    plsc.subcore_barrier()

    @pl.when(s == 0)
    def _():
        pltpu.sync_copy(h_out, hS.at[pl.ds(0, NR)])
        pltpu.sync_copy(h_in, hS.at[pl.ds(NR, NR)])

    plsc.subcore_barrier()

    @pl.when(s != 0)
    def _():
        pltpu.sync_copy(h_out, hS.at[iotab.at[0]], add=True)
        pltpu.sync_copy(h_in, hS.at[iotab.at[1]], add=True)

    plsc.subcore_barrier()
    pltpu.sync_copy(hS.at[pl.ds(s * RPT, RPT)],
                    degs_p.at[c, pl.ds(s * RPT, RPT)])
    pltpu.sync_copy(hS.at[pl.ds(NR + s * RPT, RPT)],
                    degd_p.at[c, pl.ds(s * RPT, RPT)])


_deg_kernel = functools.partial(
    pl.kernel,
    out_type=(
        jax.ShapeDtypeStruct((NC, NR, 128), jnp.float32),
        jax.ShapeDtypeStruct((NC, NR, 128), jnp.float32),
    ),
    mesh=_MESH,
    compiler_params=_SC_PARAMS,
    scratch_types=[
        pltpu.VMEM((EPW,), jnp.int32),
        pltpu.VMEM((NR, 128), jnp.float32),
        pltpu.VMEM((NR, 128), jnp.float32),
        pltpu.VMEM((2, NR), jnp.int32),
        pltpu.VMEM_SHARED((2 * NR, 128), jnp.float32),
    ],
)(_deg_body)


def _np1_body(degs_p, degd_p, z1, srcc, dstc, zrs, ns_o, nd_o, agg1p,
              psb, pdb, zbuf, ybuf, nsb, ndb, srcb, dstb, msg, y1S, aggS,
              gsem, ssem):
    c = lax.axis_index("c")
    s = lax.axis_index("s")
    w = _wid()
    nbase = s * RPS

    pltpu.sync_copy(degs_p.at[:, pl.ds(s * RPT, RPT)], psb)
    pltpu.sync_copy(degd_p.at[:, pl.ds(s * RPT, RPT)], pdb)
    pltpu.sync_copy(z1.at[pl.ds(nbase, RPS)], zbuf)
    pltpu.sync_copy(srcc.at[w], srcb)
    pltpu.sync_copy(dstc.at[w], dstb)

    def reduce_norm(pb, nb):
        for r in range(RPT):
            def vbody(j, _):
                acc = pb[0, r, pl.ds(j * L, L)] + pb[1, r, pl.ds(j * L, L)]
                nrm = _rsqrt_newton(jnp.maximum(acc, 1.0))
                nb[pl.ds(r * 128 + j * L, L)] = jnp.where(acc > 0, nrm, 0.0)
                return 0

            lax.fori_loop(0, 128 // L, vbody, 0, unroll=2)

    reduce_norm(psb, nsb)
    reduce_norm(pdb, ndb)

    ioa = lax.broadcasted_iota(jnp.int32, (L,), 0)

    def ybody(g, _):
        nv = g * L + ioa
        nsv = nsb[pl.ds(g * L, L)]
        for f in range(DH):
            fv = jnp.full((L,), f, jnp.int32)
            v = plsc.load_gather(zbuf, [nv, fv])
            plsc.store_scatter(ybuf, [nv, fv], v * nsv)
        return 0

    lax.fori_loop(0, RPS // L, ybody, 0, unroll=2)

    @pl.when(c == 0)
    def _():
        pltpu.sync_copy(nsb, ns_o.at[pl.ds(nbase, RPS)])
        pltpu.sync_copy(ndb, nd_o.at[pl.ds(nbase, RPS)])

    pltpu.sync_copy(ybuf, y1S.at[pl.ds(nbase, RPS)])
    pltpu.sync_copy(zrs, aggS.at[pl.ds(nbase, RPS)])
    plsc.subcore_barrier()

    def g_start(i, b):
        pltpu.make_async_copy(y1S.at[srcb.at[i]], msg.at[b], gsem).start()

    def g_wait(b):
        pltpu.make_async_copy(y1S.at[srcb.at[0]], msg.at[b], gsem).wait()

    def s_start(i, b):
        pltpu.make_async_copy(msg.at[b], aggS.at[dstb.at[i]],
                              ssem).start(add=True)

    def s_wait(b):
        pltpu.make_async_copy(msg.at[b], aggS.at[dstb.at[0]], ssem).wait()

    for b in range(NBUF):
        g_start(b, b)

    def ch(gi, _):
        i0 = gi * NBUF
        for b in range(NBUF):
            g_wait(b)
            s_start(i0 + b, b)
        for b in range(NBUF):
            s_wait(b)
            nxt = i0 + NBUF + b

            @pl.when(nxt < NCH)
            def _():
                g_start(nxt, b)

        return 0

    lax.fori_loop(0, NCH // NBUF, ch, 0)
    plsc.subcore_barrier()
    pltpu.sync_copy(aggS.at[pl.ds(nbase, RPS)],
                    agg1p.at[c, pl.ds(nbase, RPS)])


_np1_kernel = functools.partial(
    pl.kernel,
    out_type=(
        jax.ShapeDtypeStruct((NP,), jnp.float32),
        jax.ShapeDtypeStruct((NP,), jnp.float32),
        jax.ShapeDtypeStruct((NC, NP, DP), jnp.float32),
    ),
    mesh=_MESH,
    compiler_params=_SC_PARAMS,
    scratch_types=[
        pltpu.VMEM((NC, RPT, 128), jnp.float32),
        pltpu.VMEM((NC, RPT, 128), jnp.float32),
        pltpu.VMEM((RPS, ZDP), jnp.float32),
        pltpu.VMEM((RPS, DP), jnp.float32),
        pltpu.VMEM((RPS,), jnp.float32),
        pltpu.VMEM((RPS,), jnp.float32),
        pltpu.VMEM((NCH, ECH), jnp.int32),
        pltpu.VMEM((NCH, ECH), jnp.int32),
        pltpu.VMEM((NBUF, ECH, DP), jnp.float32),
        pltpu.VMEM_SHARED((NP, DP), jnp.float32),
        pltpu.VMEM_SHARED((NP, DP), jnp.float32),
        pltpu.SemaphoreType.DMA,
        pltpu.SemaphoreType.DMA,
    ],
)(_np1_body)


def _ep2_body(agg1p, ns_i, nd_i, b1p, w2p, srcw, dstw, agg2p,
              p0b, p1b, nsb, ndb, b1b, w2b, y2s, y2full, a2b, sb, db, iotab,
              y2S, agg2S):
    c = lax.axis_index("c")
    s = lax.axis_index("s")
    w = _wid()
    nbase = s * RPS

    pltpu.sync_copy(agg1p.at[0, pl.ds(nbase, RPS)], p0b)
    pltpu.sync_copy(agg1p.at[1, pl.ds(nbase, RPS)], p1b)
    pltpu.sync_copy(ns_i.at[pl.ds(nbase, RPS)], nsb)
    pltpu.sync_copy(nd_i.at[pl.ds(nbase, RPS)], ndb)
    pltpu.sync_copy(b1p, b1b)
    pltpu.sync_copy(w2p, w2b)
    pltpu.sync_copy(srcw.at[w], sb)
    pltpu.sync_copy(dstw.at[w], db)
    _fill_iota_row(iotab, 0, 0)

    b1v = b1b[...]
    w2v = w2b[...]
    ioa = lax.broadcasted_iota(jnp.int32, (L,), 0)

    def nbody(g, _):
        nv = g * L + ioa
        ndv = ndb[pl.ds(g * L, L)]
        nsv = nsb[pl.ds(g * L, L)]
        acc = jnp.zeros((L,), jnp.float32)
        for f in range(DH):
            fv = jnp.full((L,), f, jnp.int32)
            a = plsc.load_gather(p0b, [nv, fv]) + plsc.load_gather(p1b, [nv, fv])
            h = jnp.maximum(a * ndv + b1v[f], 0.0)
            acc = acc + h * w2v[f]
        y2s[pl.ds(g * L, L)] = acc * nsv
        return 0

    lax.fori_loop(0, RPS // L, nbody, 0, unroll=2)
    pltpu.sync_copy(y2s, y2S.at[pl.ds(nbase, RPS)])

    zero = jnp.zeros((L,), jnp.float32)

    def zbody(r, _):
        for k in range(128 // L):
            a2b[r, pl.ds(k * L, L)] = zero
        return 0

    lax.fori_loop(0, NR, zbody, 0, unroll=2)
    plsc.subcore_barrier()
    pltpu.sync_copy(y2S, y2full)

    def body(i, _):
        sv = sb[pl.ds(i * L, L)]
        dv = db[pl.ds(i * L, L)]
        v = plsc.load_gather(y2full, [sv])
        plsc.addupdate_scatter(a2b, [dv >> 7, dv & 127], v)
        return 0

    lax.fori_loop(0, EPW // L, body, 0, unroll=8)

    plsc.subcore_barrier()

    @pl.when(s == 0)
    def _():
        pltpu.sync_copy(a2b, agg2S)

    plsc.subcore_barrier()

    @pl.when(s != 0)
    def _():
        pltpu.sync_copy(a2b, agg2S.at[iotab.at[0]], add=True)

    plsc.subcore_barrier()
    pltpu.sync_copy(agg2S.at[pl.ds(s * RPT, RPT)],
                    agg2p.at[c, pl.ds(s * RPT, RPT)])


_ep2_kernel = functools.partial(
    pl.kernel,
    out_type=jax.ShapeDtypeStruct((NC, NR, 128), jnp.float32),
    mesh=_MESH,
    compiler_params=_SC_PARAMS,
    scratch_types=[
        pltpu.VMEM((RPS, DP), jnp.float32),
        pltpu.VMEM((RPS, DP), jnp.float32),
        pltpu.VMEM((RPS,), jnp.float32),
        pltpu.VMEM((RPS,), jnp.float32),
        pltpu.VMEM((L,), jnp.float32),
        pltpu.VMEM((L,), jnp.float32),
        pltpu.VMEM((RPS,), jnp.float32),
        pltpu.VMEM((NP,), jnp.float32),
        pltpu.VMEM((NR, 128), jnp.float32),
        pltpu.VMEM((EPW,), jnp.int32),
        pltpu.VMEM((EPW,), jnp.int32),
        pltpu.VMEM((2, NR), jnp.int32),
        pltpu.VMEM_SHARED((NP,), jnp.float32),
        pltpu.VMEM_SHARED((NR, 128), jnp.float32),
    ],
)(_ep2_body)


def _fin_body(agg2p, nd_i, b2p, outv, pb, ndb, b2b, ob):
    w = _wid()
    base = w * NPW
    rbase = base // 128
    pltpu.sync_copy(agg2p.at[:, pl.ds(rbase, NPW // 128)], pb)
    pltpu.sync_copy(nd_i.at[pl.ds(base, NPW)], ndb)
    pltpu.sync_copy(b2p, b2b)
    b2v = b2b[...]

    def body(r, _):
        for k in range(128 // L):
            off = r * 128 + k * L
            acc = pb[0, r, pl.ds(k * L, L)] + pb[1, r, pl.ds(k * L, L)]
            ob[pl.ds(off, L)] = acc * ndb[pl.ds(off, L)] + b2v
        return 0

    lax.fori_loop(0, NPW // 128, body, 0)
    pltpu.sync_copy(ob, outv.at[pl.ds(base, NPW)])


_fin_kernel = functools.partial(
    pl.kernel,
    out_type=jax.ShapeDtypeStruct((NP,), jnp.float32),
    mesh=_MESH,
    compiler_params=_SC_PARAMS,
    scratch_types=[
        pltpu.VMEM((NC, NPW // 128, 128), jnp.float32),
        pltpu.VMEM((NPW,), jnp.float32),
        pltpu.VMEM((L,), jnp.float32),
        pltpu.VMEM((NPW,), jnp.float32),
    ],
)(_fin_body)


def kernel(inputs, edge_index, W1, b1, W2, b2):
    x = inputs
    ei = edge_index.astype(jnp.int32)
    src, dst = ei[0], ei[1]
    xp = jnp.pad(x, ((0, NP - N), (0, 0)))
    w1p = jnp.pad(W1, ((0, 0), (0, ZDP - DH)))
    b1p = jnp.pad(b1, (0, L - DH))
    w2p = jnp.pad(W2[:, 0], (0, L - DH))
    b2p = jnp.broadcast_to(b2, (L,))
    zrs = jnp.zeros((RPS, DP), jnp.float32)
    srcw = src.reshape(NW, EPW)
    dstw = dst.reshape(NW, EPW)
    srcc = src.reshape(NW, NCH, ECH)
    dstc = dst.reshape(NW, NCH, ECH)

    z1 = _matmul(xp, w1p)
    degs_p, degd_p = _deg_kernel(srcw, dstw)
    ns, nd, agg1p = _np1_kernel(degs_p, degd_p, z1, srcc, dstc, zrs)
    agg2p = _ep2_kernel(agg1p, ns, nd, b1p, w2p, srcw, dstw)
    outv = _fin_kernel(agg2p, nd, b2p)
    return outv[:N].reshape(N, 1)


# speedup vs baseline: 1.1550x; 1.1550x over previous
import functools

import jax
import jax.numpy as jnp
from jax import lax
from jax.experimental import pallas as pl
from jax.experimental.pallas import tpu as pltpu
from jax.experimental.pallas import tpu_sc as plsc

N = 10000
NP = 12288
E = 320000
D_IN = 128
DH = 8
DP = 8
ZDP = 16
NC, NS, L = 2, 16, 16
NW = NC * NS
EPW = E // NW
ECH = 128
NCH = 80
EPWP = NCH * ECH
NPW = NP // NW
RPS = NP // NS
NR = NP // 128
RPT = NR // NS
NBUF = 4

_MESH = plsc.VectorSubcoreMesh(core_axis_name="c", subcore_axis_name="s")
_SC_PARAMS = pltpu.CompilerParams(needs_layout_passes=False,
                                  use_tc_tiling_on_sc=False)


def _wid():
    return lax.axis_index("s") * NC + lax.axis_index("c")


def _rsqrt_newton(x):
    i = lax.bitcast_convert_type(x, jnp.int32)
    y = lax.bitcast_convert_type(jnp.int32(0x5F3759DF) - (i >> 1), jnp.float32)
    for _ in range(3):
        y = y * (1.5 - 0.5 * x * y * y)
    return y


def _fill_iota_row(ref, row, offset):
    ioa = lax.broadcasted_iota(jnp.int32, (L,), 0)
    for k in range(NR // L):
        ref[row, pl.ds(k * L, L)] = ioa + (offset + k * L)


def _mm_body(x_ref, w_ref, o_ref):
    o_ref[pl.ds(0, N), :] = jnp.dot(x_ref[...], w_ref[...],
                                    preferred_element_type=jnp.float32)
    o_ref[pl.ds(N, NP - N), :] = jnp.zeros((NP - N, ZDP), jnp.float32)


def _matmul(xp, w1p):
    return pl.pallas_call(
        _mm_body,
        out_shape=jax.ShapeDtypeStruct((NP, ZDP), jnp.float32),
    )(xp, w1p)


def _deg_body(eic, degs_p, degd_p, idxv, h_out, h_in, iotab, hS):
    c = lax.axis_index("c")
    s = lax.axis_index("s")
    w = _wid()
    zero = jnp.zeros((L,), jnp.float32)

    def zbody(r, _):
        for k in range(128 // L):
            h_out[r, pl.ds(k * L, L)] = zero
            h_in[r, pl.ds(k * L, L)] = zero
        return 0

    lax.fori_loop(0, NR, zbody, 0, unroll=2)
    _fill_iota_row(iotab, 0, 0)
    _fill_iota_row(iotab, 1, NR)

    one = jnp.full((L,), 1.0, jnp.float32)
    pltpu.sync_copy(eic.at[0, w], idxv)

    def sbody(i, _):
        for k in range(ECH // L):
            d = idxv[i, pl.ds(k * L, L)]
            plsc.addupdate_scatter(h_out, [d >> 7, d & 127], one)
        return 0

    lax.fori_loop(0, NCH, sbody, 0)
    pltpu.sync_copy(eic.at[1, w], idxv)

    def dbody(i, _):
        for k in range(ECH // L):
            d = idxv[i, pl.ds(k * L, L)]
            plsc.addupdate_scatter(h_in, [d >> 7, d & 127], one)
        return 0

    lax.fori_loop(0, NCH, dbody, 0)

    plsc.subcore_barrier()

    @pl.when(s == 0)
    def _():
        pltpu.sync_copy(h_out, hS.at[pl.ds(0, NR)])
        pltpu.sync_copy(h_in, hS.at[pl.ds(NR, NR)])

    plsc.subcore_barrier()

    @pl.when(s != 0)
    def _():
        pltpu.sync_copy(h_out, hS.at[iotab.at[0]], add=True)
        pltpu.sync_copy(h_in, hS.at[iotab.at[1]], add=True)

    plsc.subcore_barrier()
    pltpu.sync_copy(hS.at[pl.ds(s * RPT, RPT)],
                    degs_p.at[c, pl.ds(s * RPT, RPT)])
    pltpu.sync_copy(hS.at[pl.ds(NR + s * RPT, RPT)],
                    degd_p.at[c, pl.ds(s * RPT, RPT)])


_deg_kernel = functools.partial(
    pl.kernel,
    out_type=(
        jax.ShapeDtypeStruct((NC, NR, 128), jnp.float32),
        jax.ShapeDtypeStruct((NC, NR, 128), jnp.float32),
    ),
    mesh=_MESH,
    compiler_params=_SC_PARAMS,
    scratch_types=[
        pltpu.VMEM((NCH, ECH), jnp.int32),
        pltpu.VMEM((NR, 128), jnp.float32),
        pltpu.VMEM((NR, 128), jnp.float32),
        pltpu.VMEM((2, NR), jnp.int32),
        pltpu.VMEM_SHARED((2 * NR, 128), jnp.float32),
    ],
)(_deg_body)


def _np1_body(degs_p, degd_p, z1, eic, zrs, ns_o, nd_o, agg1p,
              psb, pdb, zbuf, ybuf, nsb, ndb, srcb, dstb, msg, y1S, aggS,
              gsem, ssem):
    c = lax.axis_index("c")
    s = lax.axis_index("s")
    w = _wid()
    nbase = s * RPS

    pltpu.sync_copy(degs_p.at[:, pl.ds(s * RPT, RPT)], psb)
    pltpu.sync_copy(degd_p.at[:, pl.ds(s * RPT, RPT)], pdb)
    pltpu.sync_copy(z1.at[pl.ds(nbase, RPS)], zbuf)
    pltpu.sync_copy(eic.at[0, w], srcb)
    pltpu.sync_copy(eic.at[1, w], dstb)

    def reduce_norm(pb, nb):
        for r in range(RPT):
            def vbody(j, _):
                acc = pb[0, r, pl.ds(j * L, L)] + pb[1, r, pl.ds(j * L, L)]
                nrm = _rsqrt_newton(jnp.maximum(acc, 1.0))
                nb[pl.ds(r * 128 + j * L, L)] = jnp.where(acc > 0, nrm, 0.0)
                return 0

            lax.fori_loop(0, 128 // L, vbody, 0, unroll=2)

    reduce_norm(psb, nsb)
    reduce_norm(pdb, ndb)

    ioa = lax.broadcasted_iota(jnp.int32, (L,), 0)

    def ybody(g, _):
        nv = g * L + ioa
        nsv = nsb[pl.ds(g * L, L)]
        for f in range(DH):
            fv = jnp.full((L,), f, jnp.int32)
            v = plsc.load_gather(zbuf, [nv, fv])
            plsc.store_scatter(ybuf, [nv, fv],
                               jnp.where(nsv > 0.0, v * nsv, 0.0))
        return 0

    lax.fori_loop(0, RPS // L, ybody, 0, unroll=2)

    @pl.when(c == 0)
    def _():
        pltpu.sync_copy(nsb, ns_o.at[pl.ds(nbase, RPS)])
        pltpu.sync_copy(ndb, nd_o.at[pl.ds(nbase, RPS)])

    pltpu.sync_copy(ybuf, y1S.at[pl.ds(nbase, RPS)])
    pltpu.sync_copy(zrs, aggS.at[pl.ds(nbase, RPS)])
    plsc.subcore_barrier()

    def g_start(i, b):
        pltpu.make_async_copy(y1S.at[srcb.at[i]], msg.at[b], gsem).start()

    def g_wait(b):
        pltpu.make_async_copy(y1S.at[srcb.at[0]], msg.at[b], gsem).wait()

    def s_start(i, b):
        pltpu.make_async_copy(msg.at[b], aggS.at[dstb.at[i]],
                              ssem).start(add=True)

    def s_wait(b):
        pltpu.make_async_copy(msg.at[b], aggS.at[dstb.at[0]], ssem).wait()

    for b in range(NBUF):
        g_start(b, b)

    def ch(gi, _):
        i0 = gi * NBUF
        for b in range(NBUF):
            g_wait(b)
            s_start(i0 + b, b)
        for b in range(NBUF):
            s_wait(b)
            nxt = i0 + NBUF + b

            @pl.when(nxt < NCH)
            def _():
                g_start(nxt, b)

        return 0

    lax.fori_loop(0, NCH // NBUF, ch, 0)
    plsc.subcore_barrier()
    pltpu.sync_copy(aggS.at[pl.ds(nbase, RPS)],
                    agg1p.at[c, pl.ds(nbase, RPS)])


_np1_kernel = functools.partial(
    pl.kernel,
    out_type=(
        jax.ShapeDtypeStruct((NP,), jnp.float32),
        jax.ShapeDtypeStruct((NP,), jnp.float32),
        jax.ShapeDtypeStruct((NC, NP, DP), jnp.float32),
    ),
    mesh=_MESH,
    compiler_params=_SC_PARAMS,
    scratch_types=[
        pltpu.VMEM((NC, RPT, 128), jnp.float32),
        pltpu.VMEM((NC, RPT, 128), jnp.float32),
        pltpu.VMEM((RPS, ZDP), jnp.float32),
        pltpu.VMEM((RPS, DP), jnp.float32),
        pltpu.VMEM((RPS,), jnp.float32),
        pltpu.VMEM((RPS,), jnp.float32),
        pltpu.VMEM((NCH, ECH), jnp.int32),
        pltpu.VMEM((NCH, ECH), jnp.int32),
        pltpu.VMEM((NBUF, ECH, DP), jnp.float32),
        pltpu.VMEM_SHARED((NP, DP), jnp.float32),
        pltpu.VMEM_SHARED((NP, DP), jnp.float32),
        pltpu.SemaphoreType.DMA,
        pltpu.SemaphoreType.DMA,
    ],
)(_np1_body)


def _ep2_body(agg1p, ns_i, nd_i, b1p, w2p, eic, agg2p,
              p0b, p1b, nsb, ndb, b1b, w2b, y2s, y2full, a2b, sb, db, iotab,
              y2S, agg2S):
    c = lax.axis_index("c")
    s = lax.axis_index("s")
    w = _wid()
    nbase = s * RPS

    pltpu.sync_copy(agg1p.at[0, pl.ds(nbase, RPS)], p0b)
    pltpu.sync_copy(agg1p.at[1, pl.ds(nbase, RPS)], p1b)
    pltpu.sync_copy(ns_i.at[pl.ds(nbase, RPS)], nsb)
    pltpu.sync_copy(nd_i.at[pl.ds(nbase, RPS)], ndb)
    pltpu.sync_copy(b1p, b1b)
    pltpu.sync_copy(w2p, w2b)
    pltpu.sync_copy(eic.at[0, w], sb)
    pltpu.sync_copy(eic.at[1, w], db)
    _fill_iota_row(iotab, 0, 0)

    b1v = b1b[...]
    w2v = w2b[...]
    ioa = lax.broadcasted_iota(jnp.int32, (L,), 0)

    def nbody(g, _):
        nv = g * L + ioa
        ndv = ndb[pl.ds(g * L, L)]
        nsv = nsb[pl.ds(g * L, L)]
        acc = jnp.zeros((L,), jnp.float32)
        for f in range(DH):
            fv = jnp.full((L,), f, jnp.int32)
            a = plsc.load_gather(p0b, [nv, fv]) + plsc.load_gather(p1b, [nv, fv])
            h = jnp.maximum(a * ndv + b1v[f], 0.0)
            acc = acc + h * w2v[f]
        y2s[pl.ds(g * L, L)] = acc * nsv
        return 0

    lax.fori_loop(0, RPS // L, nbody, 0, unroll=2)
    pltpu.sync_copy(y2s, y2S.at[pl.ds(nbase, RPS)])

    zero = jnp.zeros((L,), jnp.float32)

    def zbody(r, _):
        for k in range(128 // L):
            a2b[r, pl.ds(k * L, L)] = zero
        return 0

    lax.fori_loop(0, NR, zbody, 0, unroll=2)
    plsc.subcore_barrier()
    pltpu.sync_copy(y2S, y2full)

    def body(i, _):
        for k in range(ECH // L):
            sv = sb[i, pl.ds(k * L, L)]
            dv = db[i, pl.ds(k * L, L)]
            v = plsc.load_gather(y2full, [sv])
            plsc.addupdate_scatter(a2b, [dv >> 7, dv & 127], v)
        return 0

    lax.fori_loop(0, NCH, body, 0)

    plsc.subcore_barrier()

    @pl.when(s == 0)
    def _():
        pltpu.sync_copy(a2b, agg2S)

    plsc.subcore_barrier()

    @pl.when(s != 0)
    def _():
        pltpu.sync_copy(a2b, agg2S.at[iotab.at[0]], add=True)

    plsc.subcore_barrier()
    pltpu.sync_copy(agg2S.at[pl.ds(s * RPT, RPT)],
                    agg2p.at[c, pl.ds(s * RPT, RPT)])


_ep2_kernel = functools.partial(
    pl.kernel,
    out_type=jax.ShapeDtypeStruct((NC, NR, 128), jnp.float32),
    mesh=_MESH,
    compiler_params=_SC_PARAMS,
    scratch_types=[
        pltpu.VMEM((RPS, DP), jnp.float32),
        pltpu.VMEM((RPS, DP), jnp.float32),
        pltpu.VMEM((RPS,), jnp.float32),
        pltpu.VMEM((RPS,), jnp.float32),
        pltpu.VMEM((L,), jnp.float32),
        pltpu.VMEM((L,), jnp.float32),
        pltpu.VMEM((RPS,), jnp.float32),
        pltpu.VMEM((NP,), jnp.float32),
        pltpu.VMEM((NR, 128), jnp.float32),
        pltpu.VMEM((NCH, ECH), jnp.int32),
        pltpu.VMEM((NCH, ECH), jnp.int32),
        pltpu.VMEM((2, NR), jnp.int32),
        pltpu.VMEM_SHARED((NP,), jnp.float32),
        pltpu.VMEM_SHARED((NR, 128), jnp.float32),
    ],
)(_ep2_body)


def _fin_body(agg2p, nd_i, b2p, outv, pb, ndb, b2b, ob):
    w = _wid()
    base = w * NPW
    rbase = base // 128
    pltpu.sync_copy(agg2p.at[:, pl.ds(rbase, NPW // 128)], pb)
    pltpu.sync_copy(nd_i.at[pl.ds(base, NPW)], ndb)
    pltpu.sync_copy(b2p, b2b)
    b2v = b2b[...]

    def body(r, _):
        for k in range(128 // L):
            off = r * 128 + k * L
            acc = pb[0, r, pl.ds(k * L, L)] + pb[1, r, pl.ds(k * L, L)]
            ob[pl.ds(off, L)] = acc * ndb[pl.ds(off, L)] + b2v
        return 0

    lax.fori_loop(0, NPW // 128, body, 0)
    pltpu.sync_copy(ob, outv.at[pl.ds(base, NPW)])


_fin_kernel = functools.partial(
    pl.kernel,
    out_type=jax.ShapeDtypeStruct((NP,), jnp.float32),
    mesh=_MESH,
    compiler_params=_SC_PARAMS,
    scratch_types=[
        pltpu.VMEM((NC, NPW // 128, 128), jnp.float32),
        pltpu.VMEM((NPW,), jnp.float32),
        pltpu.VMEM((L,), jnp.float32),
        pltpu.VMEM((NPW,), jnp.float32),
    ],
)(_fin_body)


def kernel(inputs, edge_index, W1, b1, W2, b2):
    x = inputs
    ei = edge_index.astype(jnp.int32)
    w1p = jnp.pad(W1, ((0, 0), (0, ZDP - DH)))
    b1p = jnp.pad(b1, (0, L - DH))
    w2p = jnp.pad(W2[:, 0], (0, L - DH))
    b2p = jnp.broadcast_to(b2, (L,))
    zrs = jnp.zeros((RPS, DP), jnp.float32)
    eic = jnp.pad(ei.reshape(2, NW, EPW), ((0, 0), (0, 0), (0, EPWP - EPW)),
                  constant_values=NP - 1).reshape(2, NW, NCH, ECH)

    z1 = _matmul(x, w1p)
    degs_p, degd_p = _deg_kernel(eic)
    ns, nd, agg1p = _np1_kernel(degs_p, degd_p, z1, eic, zrs)
    agg2p = _ep2_kernel(agg1p, ns, nd, b1p, w2p, eic)
    outv = _fin_kernel(agg2p, nd, b2p)
    return outv[:N].reshape(N, 1)


# speedup vs baseline: 1.1904x; 1.0306x over previous
import functools

import jax
import jax.numpy as jnp
from jax import lax
from jax.experimental import pallas as pl
from jax.experimental.pallas import tpu as pltpu
from jax.experimental.pallas import tpu_sc as plsc

N = 10000
NP = 12288
E = 320000
D_IN = 128
DH = 8
DP = 8
ZDP = 16
NC, NS, L = 2, 16, 16
NW = NC * NS
EPW = E // NW
ECH = 128
NCH = 80
EPWP = NCH * ECH
NPW = NP // NW
RPS = NP // NS
NR = NP // 128
RPT = NR // NS
NBUF = 4

_MESH = plsc.VectorSubcoreMesh(core_axis_name="c", subcore_axis_name="s")
_SC_PARAMS = pltpu.CompilerParams(needs_layout_passes=False,
                                  use_tc_tiling_on_sc=False)


def _wid():
    return lax.axis_index("s") * NC + lax.axis_index("c")


def _rsqrt_newton(x):
    i = lax.bitcast_convert_type(x, jnp.int32)
    y = lax.bitcast_convert_type(jnp.int32(0x5F3759DF) - (i >> 1), jnp.float32)
    for _ in range(3):
        y = y * (1.5 - 0.5 * x * y * y)
    return y


def _fill_iota_row(ref, row, offset):
    ioa = lax.broadcasted_iota(jnp.int32, (L,), 0)
    for k in range(NR // L):
        ref[row, pl.ds(k * L, L)] = ioa + (offset + k * L)


def _mm_body(x_ref, w_ref, o_ref):
    o_ref[pl.ds(0, N), :] = jnp.dot(x_ref[...], w_ref[...],
                                    preferred_element_type=jnp.float32)
    o_ref[pl.ds(N, NP - N), :] = jnp.zeros((NP - N, ZDP), jnp.float32)


def _matmul(xp, w1p):
    return pl.pallas_call(
        _mm_body,
        out_shape=jax.ShapeDtypeStruct((NP, ZDP), jnp.float32),
    )(xp, w1p)


def _deg_body(ei, degs_p, degd_p, idxv, h_out, h_in, pcol, hSo, hSi):
    c = lax.axis_index("c")
    s = lax.axis_index("s")
    w = _wid()
    nbase = s * RPS
    zero = jnp.zeros((L,), jnp.float32)

    def zbody(i, _):
        h_out[pl.ds(i * L, L)] = zero
        h_in[pl.ds(i * L, L)] = zero
        return 0

    lax.fori_loop(0, NP // L, zbody, 0, unroll=8)

    one = jnp.full((L,), 1.0, jnp.float32)
    pltpu.sync_copy(ei.at[0, pl.ds(w * EPW, EPW)], idxv)

    def sbody(i, _):
        plsc.addupdate_scatter(h_out, [idxv[pl.ds(i * L, L)]], one)
        return 0

    lax.fori_loop(0, EPW // L, sbody, 0, unroll=8)
    pltpu.sync_copy(ei.at[1, pl.ds(w * EPW, EPW)], idxv)

    def dbody(i, _):
        plsc.addupdate_scatter(h_in, [idxv[pl.ds(i * L, L)]], one)
        return 0

    lax.fori_loop(0, EPW // L, dbody, 0, unroll=8)

    pltpu.sync_copy(h_out, hSo.at[s])
    pltpu.sync_copy(h_in, hSi.at[s])
    plsc.subcore_barrier()

    for hS, out in ((hSo, degs_p), (hSi, degd_p)):
        pltpu.sync_copy(hS.at[:, pl.ds(nbase, RPS)], pcol)

        def cbody(j, _):
            acc = pcol[0, pl.ds(j * L, L)]
            for k in range(1, NS):
                acc = acc + pcol[k, pl.ds(j * L, L)]
            h_out[pl.ds(j * L, L)] = acc
            return 0

        lax.fori_loop(0, RPS // L, cbody, 0, unroll=2)
        pltpu.sync_copy(h_out.at[pl.ds(0, RPS)], out.at[c, pl.ds(nbase, RPS)])


_deg_kernel = functools.partial(
    pl.kernel,
    out_type=(
        jax.ShapeDtypeStruct((NC, NP), jnp.float32),
        jax.ShapeDtypeStruct((NC, NP), jnp.float32),
    ),
    mesh=_MESH,
    compiler_params=_SC_PARAMS,
    scratch_types=[
        pltpu.VMEM((EPW,), jnp.int32),
        pltpu.VMEM((NP,), jnp.float32),
        pltpu.VMEM((NP,), jnp.float32),
        pltpu.VMEM((NS, RPS), jnp.float32),
        pltpu.VMEM_SHARED((NS, NP), jnp.float32),
        pltpu.VMEM_SHARED((NS, NP), jnp.float32),
    ],
)(_deg_body)


def _np1_body(degs_p, degd_p, z1, eic, zrs, ns_o, nd_o, agg1p,
              psb, pdb, zbuf, ybuf, nsb, ndb, srcb, dstb, msg, y1S, aggS,
              gsem, ssem):
    c = lax.axis_index("c")
    s = lax.axis_index("s")
    w = _wid()
    nbase = s * RPS

    pltpu.sync_copy(degs_p.at[:, pl.ds(nbase, RPS)], psb)
    pltpu.sync_copy(degd_p.at[:, pl.ds(nbase, RPS)], pdb)
    pltpu.sync_copy(z1.at[pl.ds(nbase, RPS)], zbuf)
    pltpu.sync_copy(eic.at[0, w], srcb)
    pltpu.sync_copy(eic.at[1, w], dstb)

    def reduce_norm(pb, nb):
        def vbody(j, _):
            acc = pb[0, pl.ds(j * L, L)] + pb[1, pl.ds(j * L, L)]
            nrm = _rsqrt_newton(jnp.maximum(acc, 1.0))
            nb[pl.ds(j * L, L)] = jnp.where(acc > 0, nrm, 0.0)
            return 0

        lax.fori_loop(0, RPS // L, vbody, 0, unroll=2)

    reduce_norm(psb, nsb)
    reduce_norm(pdb, ndb)

    ioa = lax.broadcasted_iota(jnp.int32, (L,), 0)

    def ybody(g, _):
        nv = g * L + ioa
        nsv = nsb[pl.ds(g * L, L)]
        for f in range(DH):
            fv = jnp.full((L,), f, jnp.int32)
            v = plsc.load_gather(zbuf, [nv, fv])
            plsc.store_scatter(ybuf, [nv, fv],
                               jnp.where(nsv > 0.0, v * nsv, 0.0))
        return 0

    lax.fori_loop(0, RPS // L, ybody, 0, unroll=2)

    @pl.when(c == 0)
    def _():
        pltpu.sync_copy(nsb, ns_o.at[pl.ds(nbase, RPS)])
        pltpu.sync_copy(ndb, nd_o.at[pl.ds(nbase, RPS)])

    pltpu.sync_copy(ybuf, y1S.at[pl.ds(nbase, RPS)])
    pltpu.sync_copy(zrs, aggS.at[pl.ds(nbase, RPS)])
    plsc.subcore_barrier()

    def g_start(i, b):
        pltpu.make_async_copy(y1S.at[srcb.at[i]], msg.at[b], gsem).start()

    def g_wait(b):
        pltpu.make_async_copy(y1S.at[srcb.at[0]], msg.at[b], gsem).wait()

    def s_start(i, b):
        pltpu.make_async_copy(msg.at[b], aggS.at[dstb.at[i]],
                              ssem).start(add=True)

    def s_wait(b):
        pltpu.make_async_copy(msg.at[b], aggS.at[dstb.at[0]], ssem).wait()

    for b in range(NBUF):
        g_start(b, b)

    def ch(gi, _):
        i0 = gi * NBUF
        for b in range(NBUF):
            g_wait(b)
            s_start(i0 + b, b)
        for b in range(NBUF):
            s_wait(b)
            nxt = i0 + NBUF + b

            @pl.when(nxt < NCH)
            def _():
                g_start(nxt, b)

        return 0

    lax.fori_loop(0, NCH // NBUF, ch, 0)
    plsc.subcore_barrier()
    pltpu.sync_copy(aggS.at[pl.ds(nbase, RPS)],
                    agg1p.at[c, pl.ds(nbase, RPS)])


_np1_kernel = functools.partial(
    pl.kernel,
    out_type=(
        jax.ShapeDtypeStruct((NP,), jnp.float32),
        jax.ShapeDtypeStruct((NP,), jnp.float32),
        jax.ShapeDtypeStruct((NC, NP, DP), jnp.float32),
    ),
    mesh=_MESH,
    compiler_params=_SC_PARAMS,
    scratch_types=[
        pltpu.VMEM((NC, RPS), jnp.float32),
        pltpu.VMEM((NC, RPS), jnp.float32),
        pltpu.VMEM((RPS, ZDP), jnp.float32),
        pltpu.VMEM((RPS, DP), jnp.float32),
        pltpu.VMEM((RPS,), jnp.float32),
        pltpu.VMEM((RPS,), jnp.float32),
        pltpu.VMEM((NCH, ECH), jnp.int32),
        pltpu.VMEM((NCH, ECH), jnp.int32),
        pltpu.VMEM((NBUF, ECH, DP), jnp.float32),
        pltpu.VMEM_SHARED((NP, DP), jnp.float32),
        pltpu.VMEM_SHARED((NP, DP), jnp.float32),
        pltpu.SemaphoreType.DMA,
        pltpu.SemaphoreType.DMA,
    ],
)(_np1_body)


def _ep2_body(agg1p, ns_i, nd_i, b1p, w2p, ei, agg2p,
              p0b, p1b, nsb, ndb, b1b, w2b, y2s, y2full, a2b, sb, db, pcol,
              y2S, a2S):
    c = lax.axis_index("c")
    s = lax.axis_index("s")
    w = _wid()
    nbase = s * RPS

    pltpu.sync_copy(agg1p.at[0, pl.ds(nbase, RPS)], p0b)
    pltpu.sync_copy(agg1p.at[1, pl.ds(nbase, RPS)], p1b)
    pltpu.sync_copy(ns_i.at[pl.ds(nbase, RPS)], nsb)
    pltpu.sync_copy(nd_i.at[pl.ds(nbase, RPS)], ndb)
    pltpu.sync_copy(b1p, b1b)
    pltpu.sync_copy(w2p, w2b)
    pltpu.sync_copy(ei.at[0, pl.ds(w * EPW, EPW)], sb)
    pltpu.sync_copy(ei.at[1, pl.ds(w * EPW, EPW)], db)

    b1v = b1b[...]
    w2v = w2b[...]
    ioa = lax.broadcasted_iota(jnp.int32, (L,), 0)

    def nbody(g, _):
        nv = g * L + ioa
        ndv = ndb[pl.ds(g * L, L)]
        nsv = nsb[pl.ds(g * L, L)]
        acc = jnp.zeros((L,), jnp.float32)
        for f in range(DH):
            fv = jnp.full((L,), f, jnp.int32)
            a = plsc.load_gather(p0b, [nv, fv]) + plsc.load_gather(p1b, [nv, fv])
            h = jnp.maximum(a * ndv + b1v[f], 0.0)
            acc = acc + h * w2v[f]
        y2s[pl.ds(g * L, L)] = acc * nsv
        return 0

    lax.fori_loop(0, RPS // L, nbody, 0, unroll=2)
    pltpu.sync_copy(y2s, y2S.at[pl.ds(nbase, RPS)])

    zero = jnp.zeros((L,), jnp.float32)

    def zbody(i, _):
        a2b[pl.ds(i * L, L)] = zero
        return 0

    lax.fori_loop(0, NP // L, zbody, 0, unroll=8)
    plsc.subcore_barrier()
    pltpu.sync_copy(y2S, y2full)

    def body(i, _):
        v = plsc.load_gather(y2full, [sb[pl.ds(i * L, L)]])
        plsc.addupdate_scatter(a2b, [db[pl.ds(i * L, L)]], v)
        return 0

    lax.fori_loop(0, EPW // L, body, 0, unroll=8)

    pltpu.sync_copy(a2b, a2S.at[s])
    plsc.subcore_barrier()
    pltpu.sync_copy(a2S.at[:, pl.ds(nbase, RPS)], pcol)

    def cbody(j, _):
        acc = pcol[0, pl.ds(j * L, L)]
        for k in range(1, NS):
            acc = acc + pcol[k, pl.ds(j * L, L)]
        a2b[pl.ds(j * L, L)] = acc
        return 0

    lax.fori_loop(0, RPS // L, cbody, 0, unroll=2)
    pltpu.sync_copy(a2b.at[pl.ds(0, RPS)], agg2p.at[c, pl.ds(nbase, RPS)])


_ep2_kernel = functools.partial(
    pl.kernel,
    out_type=jax.ShapeDtypeStruct((NC, NP), jnp.float32),
    mesh=_MESH,
    compiler_params=_SC_PARAMS,
    scratch_types=[
        pltpu.VMEM((RPS, DP), jnp.float32),
        pltpu.VMEM((RPS, DP), jnp.float32),
        pltpu.VMEM((RPS,), jnp.float32),
        pltpu.VMEM((RPS,), jnp.float32),
        pltpu.VMEM((L,), jnp.float32),
        pltpu.VMEM((L,), jnp.float32),
        pltpu.VMEM((RPS,), jnp.float32),
        pltpu.VMEM((NP,), jnp.float32),
        pltpu.VMEM((NP,), jnp.float32),
        pltpu.VMEM((EPW,), jnp.int32),
        pltpu.VMEM((EPW,), jnp.int32),
        pltpu.VMEM((NS, RPS), jnp.float32),
        pltpu.VMEM_SHARED((NP,), jnp.float32),
        pltpu.VMEM_SHARED((NS, NP), jnp.float32),
    ],
)(_ep2_body)


def _fin_body(agg2p, nd_i, b2p, outv, pb, ndb, b2b, ob):
    w = _wid()
    base = w * NPW
    pltpu.sync_copy(agg2p.at[:, pl.ds(base, NPW)], pb)
    pltpu.sync_copy(nd_i.at[pl.ds(base, NPW)], ndb)
    pltpu.sync_copy(b2p, b2b)
    b2v = b2b[...]

    def body(j, _):
        acc = pb[0, pl.ds(j * L, L)] + pb[1, pl.ds(j * L, L)]
        ob[pl.ds(j * L, L)] = acc * ndb[pl.ds(j * L, L)] + b2v
        return 0

    lax.fori_loop(0, NPW // L, body, 0, unroll=2)
    pltpu.sync_copy(ob, outv.at[pl.ds(base, NPW)])


_fin_kernel = functools.partial(
    pl.kernel,
    out_type=jax.ShapeDtypeStruct((NP,), jnp.float32),
    mesh=_MESH,
    compiler_params=_SC_PARAMS,
    scratch_types=[
        pltpu.VMEM((NC, NPW), jnp.float32),
        pltpu.VMEM((NPW,), jnp.float32),
        pltpu.VMEM((L,), jnp.float32),
        pltpu.VMEM((NPW,), jnp.float32),
    ],
)(_fin_body)


def kernel(inputs, edge_index, W1, b1, W2, b2):
    x = inputs
    ei = edge_index.astype(jnp.int32)
    w1p = jnp.pad(W1, ((0, 0), (0, ZDP - DH)))
    b1p = jnp.pad(b1, (0, L - DH))
    w2p = jnp.pad(W2[:, 0], (0, L - DH))
    b2p = jnp.broadcast_to(b2, (L,))
    zrs = jnp.zeros((RPS, DP), jnp.float32)
    eic = jnp.pad(ei.reshape(2, NW, EPW), ((0, 0), (0, 0), (0, EPWP - EPW)),
                  constant_values=NP - 1).reshape(2, NW, NCH, ECH)

    z1 = _matmul(x, w1p)
    degs_p, degd_p = _deg_kernel(ei)
    ns, nd, agg1p = _np1_kernel(degs_p, degd_p, z1, eic, zrs)
    agg2p = _ep2_kernel(agg1p, ns, nd, b1p, w2p, ei)
    outv = _fin_kernel(agg2p, nd, b2p)
    return outv[:N].reshape(N, 1)
